# baseline (speedup 1.0000x reference)
import functools

import jax
import jax.numpy as jnp
from jax import lax
from jax.experimental import pallas as pl
from jax.experimental.pallas import tpu as pltpu
from jax.experimental.pallas import tpu_sc as plsc

NUM_EMBEDDINGS = 100
EMBED_DIM = 50
ROW_STRIDE = 65
BATCH = 16384

NUM_CORES = 2
NUM_SUBCORES = 16
NUM_WORKERS = NUM_CORES * NUM_SUBCORES
ROWS_PER_WORKER = BATCH // NUM_WORKERS
LANES = 16

GROUPS_PER_WORKER = ROWS_PER_WORKER // LANES
QUARTERS = 4
GROUPS_PER_QUARTER = GROUPS_PER_WORKER // QUARTERS
ROWS_PER_QUARTER = ROWS_PER_WORKER // QUARTERS


def kernel(table, indices):
    mesh = plsc.VectorSubcoreMesh(core_axis_name="c", subcore_axis_name="s")
    table_flat = jnp.pad(
        table, ((0, 0), (0, ROW_STRIDE - EMBED_DIM))).reshape(-1)

    @functools.partial(
        pl.kernel,
        mesh=mesh,
        out_type=jax.ShapeDtypeStruct((EMBED_DIM, BATCH), jnp.float32),
        scratch_types=[
            pltpu.VMEM((NUM_EMBEDDINGS * ROW_STRIDE,), jnp.float32),
            pltpu.VMEM((ROWS_PER_WORKER,), jnp.int32),
            pltpu.VMEM((EMBED_DIM, ROWS_PER_WORKER), jnp.float32),
            pltpu.SemaphoreType.DMA,
            [pltpu.SemaphoreType.DMA] * QUARTERS,
        ],
        compiler_params=pltpu.CompilerParams(needs_layout_passes=False),
    )
    def emb_kernel(table_hbm, idx_hbm, out_hbm, tab_v, idx_v, out_v,
                   lsem, osems):
        wid = lax.axis_index("s") * NUM_CORES + lax.axis_index("c")
        row_base = wid * ROWS_PER_WORKER
        pltpu.async_copy(table_hbm, tab_v, lsem)
        pltpu.async_copy(idx_hbm.at[pl.ds(row_base, ROWS_PER_WORKER)],
                         idx_v, lsem)
        pltpu.make_async_copy(table_hbm, tab_v, lsem).wait()
        pltpu.make_async_copy(idx_hbm.at[pl.ds(row_base, ROWS_PER_WORKER)],
                              idx_v, lsem).wait()

        for q in range(QUARTERS):
            @plsc.parallel_loop(q * GROUPS_PER_QUARTER,
                                (q + 1) * GROUPS_PER_QUARTER, unroll=8)
            def _(g):
                j0 = pl.multiple_of(g * LANES, LANES)
                iv = idx_v[pl.ds(j0, LANES)]
                bases = iv * ROW_STRIDE
                for c in range(EMBED_DIM):
                    out_v[c, pl.ds(j0, LANES)] = plsc.load_gather(
                        tab_v, [bases + c])

            pltpu.async_copy(
                out_v.at[:, pl.ds(q * ROWS_PER_QUARTER, ROWS_PER_QUARTER)],
                out_hbm.at[:, pl.ds(row_base + q * ROWS_PER_QUARTER,
                                    ROWS_PER_QUARTER)],
                osems[q])
        for q in range(QUARTERS):
            pltpu.make_async_copy(
                out_v.at[:, pl.ds(q * ROWS_PER_QUARTER, ROWS_PER_QUARTER)],
                out_hbm.at[:, pl.ds(row_base + q * ROWS_PER_QUARTER,
                                    ROWS_PER_QUARTER)],
                osems[q]).wait()

    return emb_kernel(table_flat, indices).T


# speedup vs baseline: 1.0469x; 1.0469x over previous
import functools

import jax
import jax.numpy as jnp
from jax import lax
from jax.experimental import pallas as pl
from jax.experimental.pallas import tpu as pltpu
from jax.experimental.pallas import tpu_sc as plsc

NUM_EMBEDDINGS = 100
EMBED_DIM = 50
ROW_STRIDE = 65
BATCH = 16384

NUM_CORES = 2
NUM_SUBCORES = 16
NUM_WORKERS = NUM_CORES * NUM_SUBCORES
ROWS_PER_WORKER = BATCH // NUM_WORKERS
LANES = 16

GROUPS_PER_WORKER = ROWS_PER_WORKER // LANES
QUARTERS = 4
GROUPS_PER_QUARTER = GROUPS_PER_WORKER // QUARTERS
ROWS_PER_QUARTER = ROWS_PER_WORKER // QUARTERS


def kernel(table, indices):
    mesh = plsc.VectorSubcoreMesh(core_axis_name="c", subcore_axis_name="s")
    table_t = table.T

    @functools.partial(
        pl.kernel,
        mesh=mesh,
        out_type=jax.ShapeDtypeStruct((EMBED_DIM, BATCH), jnp.float32),
        scratch_types=[
            pltpu.VMEM((EMBED_DIM, NUM_EMBEDDINGS), jnp.float32),
            pltpu.VMEM((ROWS_PER_WORKER,), jnp.int32),
            pltpu.VMEM((EMBED_DIM, ROWS_PER_WORKER), jnp.float32),
            pltpu.SemaphoreType.DMA,
            [pltpu.SemaphoreType.DMA] * QUARTERS,
        ],
        compiler_params=pltpu.CompilerParams(needs_layout_passes=False),
    )
    def emb_kernel(table_hbm, idx_hbm, out_hbm, tab_v, idx_v, out_v,
                   lsem, osems):
        wid = lax.axis_index("s") * NUM_CORES + lax.axis_index("c")
        row_base = wid * ROWS_PER_WORKER
        pltpu.async_copy(table_hbm, tab_v, lsem)
        pltpu.async_copy(idx_hbm.at[pl.ds(row_base, ROWS_PER_WORKER)],
                         idx_v, lsem)
        pltpu.make_async_copy(table_hbm, tab_v, lsem).wait()
        pltpu.make_async_copy(idx_hbm.at[pl.ds(row_base, ROWS_PER_WORKER)],
                              idx_v, lsem).wait()

        for q in range(QUARTERS):
            @plsc.parallel_loop(q * GROUPS_PER_QUARTER,
                                (q + 1) * GROUPS_PER_QUARTER, unroll=4)
            def _(g):
                j0 = pl.multiple_of(g * LANES, LANES)
                iv = idx_v[pl.ds(j0, LANES)]
                for c in range(EMBED_DIM):
                    out_v[c, pl.ds(j0, LANES)] = plsc.load_gather(
                        tab_v, [jnp.full((LANES,), c, jnp.int32), iv])

            pltpu.async_copy(
                out_v.at[:, pl.ds(q * ROWS_PER_QUARTER, ROWS_PER_QUARTER)],
                out_hbm.at[:, pl.ds(row_base + q * ROWS_PER_QUARTER,
                                    ROWS_PER_QUARTER)],
                osems[q])
        for q in range(QUARTERS):
            pltpu.make_async_copy(
                out_v.at[:, pl.ds(q * ROWS_PER_QUARTER, ROWS_PER_QUARTER)],
                out_hbm.at[:, pl.ds(row_base + q * ROWS_PER_QUARTER,
                                    ROWS_PER_QUARTER)],
                osems[q]).wait()

    return emb_kernel(table_t, indices).T


# speedup vs baseline: 1.1255x; 1.0750x over previous
import functools

import jax
import jax.numpy as jnp
from jax import lax
from jax.experimental import pallas as pl
from jax.experimental.pallas import tpu as pltpu
from jax.experimental.pallas import tpu_sc as plsc

NUM_EMBEDDINGS = 100
EMBED_DIM = 50
ROW_STRIDE = 65
BATCH = 16384

NUM_CORES = 2
NUM_SUBCORES = 16
NUM_WORKERS = NUM_CORES * NUM_SUBCORES
ROWS_PER_WORKER = BATCH // NUM_WORKERS
LANES = 16

GROUPS_PER_WORKER = ROWS_PER_WORKER // LANES
QUARTERS = 4
GROUPS_PER_QUARTER = GROUPS_PER_WORKER // QUARTERS
ROWS_PER_QUARTER = ROWS_PER_WORKER // QUARTERS


def kernel(table, indices):
    mesh = plsc.VectorSubcoreMesh(core_axis_name="c", subcore_axis_name="s")
    table_flat = jnp.pad(
        table, ((0, 0), (0, ROW_STRIDE - EMBED_DIM))).reshape(-1)

    @functools.partial(
        pl.kernel,
        mesh=mesh,
        out_type=jax.ShapeDtypeStruct((EMBED_DIM, BATCH), jnp.float32),
        scratch_types=[
            pltpu.VMEM((NUM_EMBEDDINGS * ROW_STRIDE,), jnp.float32),
            pltpu.VMEM((ROWS_PER_WORKER,), jnp.int32),
            pltpu.VMEM((EMBED_DIM, ROWS_PER_WORKER), jnp.float32),
            pltpu.SemaphoreType.DMA,
            [pltpu.SemaphoreType.DMA] * QUARTERS,
        ],
        compiler_params=pltpu.CompilerParams(needs_layout_passes=False),
    )
    def emb_kernel(table_hbm, idx_hbm, out_hbm, tab_v, idx_v, out_v,
                   lsem, osems):
        wid = lax.axis_index("s") * NUM_CORES + lax.axis_index("c")
        row_base = wid * ROWS_PER_WORKER
        pltpu.async_copy(table_hbm, tab_v, lsem)
        pltpu.async_copy(idx_hbm.at[pl.ds(row_base, ROWS_PER_WORKER)],
                         idx_v, lsem)
        pltpu.make_async_copy(table_hbm, tab_v, lsem).wait()
        pltpu.make_async_copy(idx_hbm.at[pl.ds(row_base, ROWS_PER_WORKER)],
                              idx_v, lsem).wait()

        for q in range(QUARTERS):
            @plsc.parallel_loop(q * GROUPS_PER_QUARTER,
                                (q + 1) * GROUPS_PER_QUARTER, unroll=4)
            def _(g):
                j0 = pl.multiple_of(g * LANES, LANES)
                iv = idx_v[pl.ds(j0, LANES)]
                bases = iv * ROW_STRIDE
                for c in range(EMBED_DIM):
                    out_v[c, pl.ds(j0, LANES)] = plsc.load_gather(
                        tab_v, [bases + c])

            pltpu.async_copy(
                out_v.at[:, pl.ds(q * ROWS_PER_QUARTER, ROWS_PER_QUARTER)],
                out_hbm.at[:, pl.ds(row_base + q * ROWS_PER_QUARTER,
                                    ROWS_PER_QUARTER)],
                osems[q])
        for q in range(QUARTERS):
            pltpu.make_async_copy(
                out_v.at[:, pl.ds(q * ROWS_PER_QUARTER, ROWS_PER_QUARTER)],
                out_hbm.at[:, pl.ds(row_base + q * ROWS_PER_QUARTER,
                                    ROWS_PER_QUARTER)],
                osems[q]).wait()

    return emb_kernel(table_flat, indices).T


# speedup vs baseline: 1.2364x; 1.0985x over previous
import functools

import jax
import jax.numpy as jnp
from jax import lax
from jax.experimental import pallas as pl
from jax.experimental.pallas import tpu as pltpu
from jax.experimental.pallas import tpu_sc as plsc

NUM_EMBEDDINGS = 100
EMBED_DIM = 50
ROW_STRIDE = 65
BATCH = 16384

NUM_CORES = 2
NUM_SUBCORES = 16
NUM_WORKERS = NUM_CORES * NUM_SUBCORES
ROWS_PER_WORKER = BATCH // NUM_WORKERS
LANES = 16

GROUPS_PER_WORKER = ROWS_PER_WORKER // LANES
QUARTERS = 4
GROUPS_PER_QUARTER = GROUPS_PER_WORKER // QUARTERS
ROWS_PER_QUARTER = ROWS_PER_WORKER // QUARTERS


def kernel(table, indices):
    mesh = plsc.VectorSubcoreMesh(core_axis_name="c", subcore_axis_name="s")
    table_flat = jnp.pad(
        table, ((0, 0), (0, ROW_STRIDE - EMBED_DIM))).reshape(-1)

    @functools.partial(
        pl.kernel,
        mesh=mesh,
        out_type=jax.ShapeDtypeStruct((EMBED_DIM, BATCH), jnp.float32),
        scratch_types=[
            pltpu.VMEM((NUM_EMBEDDINGS * ROW_STRIDE,), jnp.float32),
            pltpu.VMEM((ROWS_PER_WORKER,), jnp.int32),
            pltpu.VMEM((EMBED_DIM, ROWS_PER_WORKER), jnp.float32),
            pltpu.SemaphoreType.DMA,
            [pltpu.SemaphoreType.DMA] * QUARTERS,
        ],
        compiler_params=pltpu.CompilerParams(needs_layout_passes=False),
    )
    def emb_kernel(table_hbm, idx_hbm, out_hbm, tab_v, idx_v, out_v,
                   lsem, osems):
        wid = lax.axis_index("s") * NUM_CORES + lax.axis_index("c")
        row_base = wid * ROWS_PER_WORKER
        pltpu.async_copy(table_hbm, tab_v, lsem)
        pltpu.async_copy(idx_hbm.at[pl.ds(row_base, ROWS_PER_WORKER)],
                         idx_v, lsem)
        pltpu.make_async_copy(table_hbm, tab_v, lsem).wait()
        pltpu.make_async_copy(idx_hbm.at[pl.ds(row_base, ROWS_PER_WORKER)],
                              idx_v, lsem).wait()

        @pl.loop(0, QUARTERS)
        def _(q):
            qrow = pl.multiple_of(q * ROWS_PER_QUARTER, ROWS_PER_QUARTER)

            @plsc.parallel_loop(0, GROUPS_PER_QUARTER, unroll=4)
            def _(g):
                j0 = pl.multiple_of(qrow + g * LANES, LANES)
                iv = idx_v[pl.ds(j0, LANES)]
                bases = iv * ROW_STRIDE
                for c in range(EMBED_DIM):
                    out_v[c, pl.ds(j0, LANES)] = plsc.load_gather(
                        tab_v, [bases + c])

            pltpu.async_copy(
                out_v.at[:, pl.ds(qrow, ROWS_PER_QUARTER)],
                out_hbm.at[:, pl.ds(row_base + qrow, ROWS_PER_QUARTER)],
                osems[0])
        for q in range(QUARTERS):
            pltpu.make_async_copy(
                out_v.at[:, pl.ds(q * ROWS_PER_QUARTER, ROWS_PER_QUARTER)],
                out_hbm.at[:, pl.ds(row_base + q * ROWS_PER_QUARTER,
                                    ROWS_PER_QUARTER)],
                osems[0]).wait()

    return emb_kernel(table_flat, indices).T
